# baseline (speedup 1.0000x reference)
import jax
import jax.numpy as jnp
from jax import lax
from jax.experimental import pallas as pl
from jax.experimental.pallas import tpu as pltpu
from jax.experimental.pallas import tpu_sc as plsc

N_FIELDS = 26
VOCAB = 100000
D_MODEL = 64
BATCH = 4096
NC, NS, L = 2, 16, 16
NW = NC * NS
B_TOTAL = N_FIELDS * BATCH

SUP = 4096
W = 512
WPS = SUP // W
SPF = (VOCAB + SUP - 1) // SUP
NSUP = N_FIELDS * SPF
ROUNDS = (NSUP + NW - 1) // NW
TAIL0 = (VOCAB // 128) * 128
WCLAMP = TAIL0 - W
NTAIL = VOCAB - TAIL0
RING = 32
VMASK = (1 << 17) - 1


def _wait_row(out_hbm, rows_v, osem):
    pltpu.make_async_copy(
        rows_v.at[pl.ds(0, D_MODEL)],
        out_hbm.at[pl.ds(0, D_MODEL)], osem).wait()


def _body(xs_hbm, tab_hbm, tail_hbm, out_hbm,
          idx_f, tabw, tailw, hitp, subp, rows_v, sem0, sem1, osem):
    wsems = (sem0, sem1)
    wid = lax.axis_index("s") * NC + lax.axis_index("c")
    lane = lax.iota(jnp.int32, L)
    d_vecs = [lane + kk * L for kk in range(D_MODEL // L)]
    blane = lane * (1 << 17)

    def emit_rows(cnt_w, outst, fbase, gather_col):
        def drain_n(n):
            def dbody(i, c):
                _wait_row(out_hbm, rows_v, osem)
                return c
            lax.fori_loop(0, n, dbody, 0)

        def hit_body(j, c):
            p = subp[pl.ds(j, L)][0]
            vv = p & VMASK
            b = lax.shift_right_logical(p, 17)
            jr = lax.rem(j, RING)
            slot = jr * D_MODEL
            for k in range(D_MODEL // L):
                col = gather_col(vv, k)
                rows_v[pl.ds(slot + k * L, L)] = col
            pltpu.async_copy(
                rows_v.at[pl.ds(slot, D_MODEL)],
                out_hbm.at[pl.ds(fbase + b * D_MODEL, D_MODEL)], osem)

            @pl.when(jr == RING - 1)
            def _():
                drain_n(RING)

            return c

        lax.fori_loop(0, cnt_w, hit_body, 0)
        drain_n(lax.rem(cnt_w, RING))
        return outst

    def sub_scan(cnt, wlo, whi):
        def sbody(jv, c):
            p = hitp[pl.ds(jv * L, L)]
            v = p & VMASK
            m = (v >= wlo) & (v < whi)
            pc = plsc.all_reduce_population_count(m)
            plsc.store_compressed(subp.at[pl.ds(c, L)], p, mask=m)
            return c + pc[0]

        nv = lax.div(cnt + (L - 1), L)
        return lax.fori_loop(0, nv, sbody, 0)

    def wstart(f, k, s0):
        w0c = jnp.minimum(s0 + k * W, WCLAMP)
        pltpu.async_copy(
            tab_hbm.at[f, :, pl.ds(w0c, W)], tabw.at[k & 1], wsems[k & 1])
        return w0c

    def wwait(f, k, s0, w0c):
        pltpu.make_async_copy(
            tab_hbm.at[f, :, pl.ds(w0c, W)], tabw.at[k & 1],
            wsems[k & 1]).wait()

    def round_fn(sid, outst):
        f = lax.div(sid, SPF)
        si = lax.rem(sid, SPF)
        s0 = si * SUP
        fbase = f * (BATCH * D_MODEL)
        islast = si == (SPF - 1)

        pltpu.sync_copy(xs_hbm.at[pl.ds(f * BATCH, BATCH)], idx_f)
        w0c = wstart(f, 0, s0)

        def scan_body(k, c):
            v = idx_f[pl.ds(k * L, L)]
            m = lax.bitcast_convert_type(v - s0, jnp.uint32) < jnp.uint32(SUP)
            pc = plsc.all_reduce_population_count(m)
            plsc.store_compressed(
                hitp.at[pl.ds(c, L)], v + blane + (k * (L << 17)), mask=m)
            return c + pc[0]

        cnt = lax.fori_loop(0, BATCH // L, scan_body, 0)
        hitp[pl.ds(cnt, L)] = jnp.full((L,), VMASK, jnp.int32)

        for k in range(WPS):
            wlo = s0 + k * W
            whi = jnp.minimum(wlo + W, TAIL0)
            cnt_w = sub_scan(cnt, wlo, whi)
            wwait(f, k, s0, w0c)
            if k + 1 < WPS:
                w1c = wstart(f, k + 1, s0)

            def gather_win(vv, kk, _w0c=w0c, _k=k):
                vv_vec = jnp.full((L,), vv, jnp.int32) - _w0c
                return plsc.load_gather(tabw.at[_k & 1], [d_vecs[kk], vv_vec])

            outst = emit_rows(cnt_w, outst, fbase, gather_win)
            if k + 1 < WPS:
                w0c = w1c

        def tail_fn(o):
            pltpu.sync_copy(
                tail_hbm.at[pl.ds(f * (NTAIL * D_MODEL), NTAIL * D_MODEL)],
                tailw)
            cnt_t = sub_scan(cnt, TAIL0, VOCAB)

            def gather_tail(vv, kk):
                idx = (vv - TAIL0) * D_MODEL + d_vecs[kk]
                return plsc.load_gather(tailw, [idx])

            return emit_rows(cnt_t, o, fbase, gather_tail)

        return lax.cond(islast, tail_fn, lambda o: o, outst)

    def one_round(r, outst):
        sid = wid + r * NW
        return lax.cond(sid < NSUP, round_fn, lambda s, o: o, sid, outst)

    lax.fori_loop(0, ROUNDS, one_round, 0)


def kernel(xs, tables):
    xs_flat = xs.reshape(B_TOTAL)
    tab_v = jnp.swapaxes(tables, 1, 2)
    tail = tables[:, TAIL0:, :].reshape(-1)
    fn = pl.kernel(
        _body,
        mesh=plsc.VectorSubcoreMesh(core_axis_name="c", subcore_axis_name="s"),
        compiler_params=pltpu.CompilerParams(
            use_tc_tiling_on_sc=True, needs_layout_passes=False),
        out_type=jax.ShapeDtypeStruct((B_TOTAL * D_MODEL,), jnp.float32),
        scratch_types=[
            pltpu.VMEM((BATCH,), jnp.int32),
            pltpu.VMEM((2, D_MODEL, W), jnp.float32),
            pltpu.VMEM((NTAIL * D_MODEL,), jnp.float32),
            pltpu.VMEM((BATCH + L,), jnp.int32),
            pltpu.VMEM((BATCH + L,), jnp.int32),
            pltpu.VMEM((RING * D_MODEL,), jnp.float32),
            pltpu.SemaphoreType.DMA,
            pltpu.SemaphoreType.DMA,
            pltpu.SemaphoreType.DMA,
        ],
    )
    out = fn(xs_flat, tab_v, tail)
    return out.reshape(N_FIELDS, BATCH, D_MODEL)


# speedup vs baseline: 1.0081x; 1.0081x over previous
import jax
import jax.numpy as jnp
from jax import lax
from jax.experimental import pallas as pl
from jax.experimental.pallas import tpu as pltpu
from jax.experimental.pallas import tpu_sc as plsc

N_FIELDS = 26
VOCAB = 100000
D_MODEL = 64
BATCH = 4096
NC, NS, L = 2, 16, 16
NW = NC * NS
B_TOTAL = N_FIELDS * BATCH

SUP = 4096
W = 512
WPS = SUP // W
SPF = (VOCAB + SUP - 1) // SUP
NSUP = N_FIELDS * SPF
ROUNDS = (NSUP + NW - 1) // NW
TAIL0 = (VOCAB // 128) * 128
WCLAMP = TAIL0 - W
NTAIL = VOCAB - TAIL0
RING = 32
VMASK = (1 << 17) - 1


def _wait_row(out_hbm, rows_v, osem):
    pltpu.make_async_copy(
        rows_v.at[pl.ds(0, D_MODEL)],
        out_hbm.at[pl.ds(0, D_MODEL)], osem).wait()


def _body(xs_hbm, tab_hbm, tail_hbm, out_hbm,
          idx_f, tabw, tailw, hitp, subp, rows_v, sem0, sem1, osem):
    wsems = (sem0, sem1)
    wid = lax.axis_index("s") * NC + lax.axis_index("c")
    lane = lax.iota(jnp.int32, L)
    d_vecs = [lane + kk * L for kk in range(D_MODEL // L)]
    blane = lane * (1 << 17)

    def emit_rows(cnt_w, outst, fbase, gather_col):
        def drain_n(n):
            def dbody(i, c):
                _wait_row(out_hbm, rows_v, osem)
                return c
            lax.fori_loop(0, n, dbody, 0)

        def hit_body(j, c):
            p = subp[pl.ds(j, L)][0]
            vv = p & VMASK
            b = lax.shift_right_logical(p, 17)
            jr = lax.rem(j, RING)
            slot = jr * D_MODEL
            for k in range(D_MODEL // L):
                col = gather_col(vv, k)
                rows_v[pl.ds(slot + k * L, L)] = col
            pltpu.async_copy(
                rows_v.at[pl.ds(slot, D_MODEL)],
                out_hbm.at[pl.ds(fbase + b * D_MODEL, D_MODEL)], osem)

            @pl.when(jr == RING - 1)
            def _():
                drain_n(RING)

            return c

        lax.fori_loop(0, cnt_w, hit_body, 0)
        drain_n(lax.rem(cnt_w, RING))
        return outst

    def sub_scan(cnt, wlo, whi):
        def sbody(jv, c):
            p = hitp[pl.ds(jv * L, L)]
            v = p & VMASK
            m = (v >= wlo) & (v < whi)
            pc = plsc.all_reduce_population_count(m)
            plsc.store_compressed(subp.at[pl.ds(c, L)], p, mask=m)
            return c + pc[0]

        nv = lax.div(cnt + (L - 1), L)
        return lax.fori_loop(0, nv, sbody, 0)

    def wstart(f, k, s0):
        w0c = jnp.minimum(s0 + k * W, WCLAMP)
        pltpu.async_copy(
            tab_hbm.at[f, :, pl.ds(w0c, W)], tabw.at[k & 1], wsems[k & 1])
        return w0c

    def wwait(f, k, s0, w0c):
        pltpu.make_async_copy(
            tab_hbm.at[f, :, pl.ds(w0c, W)], tabw.at[k & 1],
            wsems[k & 1]).wait()

    def round_fn(sid, outst):
        f = lax.div(sid, SPF)
        si = lax.rem(sid, SPF)
        s0 = si * SUP
        fbase = f * (BATCH * D_MODEL)
        islast = si == (SPF - 1)

        pltpu.sync_copy(xs_hbm.at[pl.ds(f * BATCH, BATCH)], idx_f)
        w_pend = [wstart(f, 0, s0), wstart(f, 1, s0)]

        def scan_body(k, c):
            v = idx_f[pl.ds(k * L, L)]
            m = lax.bitcast_convert_type(v - s0, jnp.uint32) < jnp.uint32(SUP)
            pc = plsc.all_reduce_population_count(m)
            plsc.store_compressed(
                hitp.at[pl.ds(c, L)], v + blane + (k * (L << 17)), mask=m)
            return c + pc[0]

        cnt = lax.fori_loop(0, BATCH // L, scan_body, 0)
        hitp[pl.ds(cnt, L)] = jnp.full((L,), VMASK, jnp.int32)

        for k in range(WPS):
            wlo = s0 + k * W
            whi = jnp.minimum(wlo + W, TAIL0)
            cnt_w = sub_scan(cnt, wlo, whi)
            w0c = w_pend[0]
            wwait(f, k, s0, w0c)

            def gather_win(vv, kk, _w0c=w0c, _k=k):
                vv_vec = jnp.full((L,), vv, jnp.int32) - _w0c
                return plsc.load_gather(tabw.at[_k & 1], [d_vecs[kk], vv_vec])

            outst = emit_rows(cnt_w, outst, fbase, gather_win)
            w_pend = [w_pend[1],
                      wstart(f, k + 2, s0) if k + 2 < WPS else None]

        def tail_fn(o):
            pltpu.sync_copy(
                tail_hbm.at[pl.ds(f * (NTAIL * D_MODEL), NTAIL * D_MODEL)],
                tailw)
            cnt_t = sub_scan(cnt, TAIL0, VOCAB)

            def gather_tail(vv, kk):
                idx = (vv - TAIL0) * D_MODEL + d_vecs[kk]
                return plsc.load_gather(tailw, [idx])

            return emit_rows(cnt_t, o, fbase, gather_tail)

        return lax.cond(islast, tail_fn, lambda o: o, outst)

    def one_round(r, outst):
        sid = wid + r * NW
        return lax.cond(sid < NSUP, round_fn, lambda s, o: o, sid, outst)

    lax.fori_loop(0, ROUNDS, one_round, 0)


def kernel(xs, tables):
    xs_flat = xs.reshape(B_TOTAL)
    tab_v = jnp.swapaxes(tables, 1, 2)
    tail = tables[:, TAIL0:, :].reshape(-1)
    fn = pl.kernel(
        _body,
        mesh=plsc.VectorSubcoreMesh(core_axis_name="c", subcore_axis_name="s"),
        compiler_params=pltpu.CompilerParams(
            use_tc_tiling_on_sc=True, needs_layout_passes=False),
        out_type=jax.ShapeDtypeStruct((B_TOTAL * D_MODEL,), jnp.float32),
        scratch_types=[
            pltpu.VMEM((BATCH,), jnp.int32),
            pltpu.VMEM((2, D_MODEL, W), jnp.float32),
            pltpu.VMEM((NTAIL * D_MODEL,), jnp.float32),
            pltpu.VMEM((BATCH + L,), jnp.int32),
            pltpu.VMEM((BATCH + L,), jnp.int32),
            pltpu.VMEM((RING * D_MODEL,), jnp.float32),
            pltpu.SemaphoreType.DMA,
            pltpu.SemaphoreType.DMA,
            pltpu.SemaphoreType.DMA,
        ],
    )
    out = fn(xs_flat, tab_v, tail)
    return out.reshape(N_FIELDS, BATCH, D_MODEL)


# speedup vs baseline: 1.0764x; 1.0678x over previous
import jax
import jax.numpy as jnp
from jax import lax
from jax.experimental import pallas as pl
from jax.experimental.pallas import tpu as pltpu
from jax.experimental.pallas import tpu_sc as plsc

N_FIELDS = 26
VOCAB = 100000
D_MODEL = 64
BATCH = 4096
NC, NS, L = 2, 16, 16
NW = NC * NS
B_TOTAL = N_FIELDS * BATCH

SUP = 4096
W = 512
WPS = SUP // W
SPF = (VOCAB + SUP - 1) // SUP
NSUP = N_FIELDS * SPF
ROUNDS = (NSUP + NW - 1) // NW
TAIL0 = (VOCAB // 128) * 128
WCLAMP = TAIL0 - W
NTAIL = VOCAB - TAIL0
RING = 64
NBUF = 3
VMASK = (1 << 17) - 1


def _wait_row(out_hbm, rows_v, osem):
    pltpu.make_async_copy(
        rows_v.at[pl.ds(0, D_MODEL)],
        out_hbm.at[pl.ds(0, D_MODEL)], osem).wait()


def _body(xs_hbm, tab_hbm, tail_hbm, out_hbm,
          idx_f, tabw, tailw, hitp, subp, rows_v, sem0, sem1, sem2, osem):
    wsems = (sem0, sem1, sem2)
    wid = lax.axis_index("s") * NC + lax.axis_index("c")
    lane = lax.iota(jnp.int32, L)
    d_vecs = [lane + kk * L for kk in range(D_MODEL // L)]
    blane = lane * (1 << 17)

    def emit_rows(cnt_w, outst, fbase, gather_col):
        def drain_n(n):
            def dbody(i, c):
                _wait_row(out_hbm, rows_v, osem)
                return c
            lax.fori_loop(0, n, dbody, 0)

        def hit_body(j, c):
            p = subp[pl.ds(j, L)][0]
            vv = p & VMASK
            b = lax.shift_right_logical(p, 17)
            jr = lax.rem(j, RING)
            slot = jr * D_MODEL
            for k in range(D_MODEL // L):
                col = gather_col(vv, k)
                rows_v[pl.ds(slot + k * L, L)] = col
            pltpu.async_copy(
                rows_v.at[pl.ds(slot, D_MODEL)],
                out_hbm.at[pl.ds(fbase + b * D_MODEL, D_MODEL)], osem)

            @pl.when(jr == RING - 1)
            def _():
                drain_n(RING)

            return c

        lax.fori_loop(0, cnt_w, hit_body, 0)
        drain_n(lax.rem(cnt_w, RING))
        return outst

    def sub_scan(cnt, wlo, whi):
        def sbody(jv, c):
            p = hitp[pl.ds(jv * L, L)]
            v = p & VMASK
            m = (v >= wlo) & (v < whi)
            pc = plsc.all_reduce_population_count(m)
            plsc.store_compressed(subp.at[pl.ds(c, L)], p, mask=m)
            return c + pc[0]

        nv = lax.div(cnt + (L - 1), L)
        return lax.fori_loop(0, nv, sbody, 0)

    def wstart(f, k, s0):
        w0c = jnp.minimum(s0 + k * W, WCLAMP)
        pltpu.async_copy(
            tab_hbm.at[f, :, pl.ds(w0c, W)], tabw.at[k % NBUF], wsems[k % NBUF])
        return w0c

    def wwait(f, k, s0, w0c):
        pltpu.make_async_copy(
            tab_hbm.at[f, :, pl.ds(w0c, W)], tabw.at[k % NBUF],
            wsems[k % NBUF]).wait()

    def round_fn(sid, outst):
        f = lax.div(sid, SPF)
        si = lax.rem(sid, SPF)
        s0 = si * SUP
        fbase = f * (BATCH * D_MODEL)
        islast = si == (SPF - 1)

        pltpu.sync_copy(xs_hbm.at[pl.ds(f * BATCH, BATCH)], idx_f)
        w_pend = [wstart(f, 0, s0), wstart(f, 1, s0)]

        def scan_body(k, c):
            v = idx_f[pl.ds(k * L, L)]
            m = lax.bitcast_convert_type(v - s0, jnp.uint32) < jnp.uint32(SUP)
            pc = plsc.all_reduce_population_count(m)
            plsc.store_compressed(
                hitp.at[pl.ds(c, L)], v + blane + (k * (L << 17)), mask=m)
            return c + pc[0]

        cnt = lax.fori_loop(0, BATCH // L, scan_body, 0)
        hitp[pl.ds(cnt, L)] = jnp.full((L,), VMASK, jnp.int32)

        for k in range(WPS):
            wlo = s0 + k * W
            whi = jnp.minimum(wlo + W, TAIL0)
            cnt_w = sub_scan(cnt, wlo, whi)
            w0c = w_pend[0]
            wwait(f, k, s0, w0c)
            w_pend = [w_pend[1],
                      wstart(f, k + 2, s0) if k + 2 < WPS else None]

            def gather_win(vv, kk, _w0c=w0c, _k=k):
                vv_vec = jnp.full((L,), vv, jnp.int32) - _w0c
                return plsc.load_gather(tabw.at[_k % NBUF],
                                        [d_vecs[kk], vv_vec])

            outst = emit_rows(cnt_w, outst, fbase, gather_win)

        def tail_fn(o):
            pltpu.sync_copy(
                tail_hbm.at[pl.ds(f * (NTAIL * D_MODEL), NTAIL * D_MODEL)],
                tailw)
            cnt_t = sub_scan(cnt, TAIL0, VOCAB)

            def gather_tail(vv, kk):
                idx = (vv - TAIL0) * D_MODEL + d_vecs[kk]
                return plsc.load_gather(tailw, [idx])

            return emit_rows(cnt_t, o, fbase, gather_tail)

        return lax.cond(islast, tail_fn, lambda o: o, outst)

    def one_round(r, outst):
        sid = wid + r * NW
        return lax.cond(sid < NSUP, round_fn, lambda s, o: o, sid, outst)

    lax.fori_loop(0, ROUNDS, one_round, 0)


def kernel(xs, tables):
    xs_flat = xs.reshape(B_TOTAL)
    tab_v = jnp.swapaxes(tables, 1, 2)
    tail = tables[:, TAIL0:, :].reshape(-1)
    fn = pl.kernel(
        _body,
        mesh=plsc.VectorSubcoreMesh(core_axis_name="c", subcore_axis_name="s"),
        compiler_params=pltpu.CompilerParams(
            use_tc_tiling_on_sc=True, needs_layout_passes=False),
        out_type=jax.ShapeDtypeStruct((B_TOTAL * D_MODEL,), jnp.float32),
        scratch_types=[
            pltpu.VMEM((BATCH,), jnp.int32),
            pltpu.VMEM((NBUF, D_MODEL, W), jnp.float32),
            pltpu.VMEM((NTAIL * D_MODEL,), jnp.float32),
            pltpu.VMEM((BATCH + L,), jnp.int32),
            pltpu.VMEM((BATCH + L,), jnp.int32),
            pltpu.VMEM((RING * D_MODEL,), jnp.float32),
            pltpu.SemaphoreType.DMA,
            pltpu.SemaphoreType.DMA,
            pltpu.SemaphoreType.DMA,
            pltpu.SemaphoreType.DMA,
        ],
    )
    out = fn(xs_flat, tab_v, tail)
    return out.reshape(N_FIELDS, BATCH, D_MODEL)
